# Initial kernel scaffold; baseline (speedup 1.0000x reference)
import jax
import jax.numpy as jnp
from jax.experimental import pallas as pl


def kernel(x, pos, w1, b1, g1, be1, w2, b2, g2, be2, wm, bm, gm, bem, wc1, bc1, gc1, bec1, wc2, bc2, gc2, bec2, wf, bf, wl, bl):
    raise NotImplementedError("write your pallas kernel here")


# speedup vs baseline: 4.0977x; 4.0977x over previous
import functools

import jax
import jax.numpy as jnp
from jax import lax
from jax.experimental import pallas as pl
from jax.experimental.pallas import tpu as pltpu
from jax.experimental.pallas import tpu_sc as plsc

P = 4096
K = 16
D = 3
C_IN = 64
C_DELTA = 16
C_TOT = 80
C_OUT = 64
EPS = 1e-5
RB = 512
_SC_CORES = 2
_SC_SUBCORES = 16
_SC_CHUNK = 256


def _knn_body(sqr_ref, sqc_ref, augb_ref, posT_ref, aug_ref, col_ref, prel_ref):
    pr = augb_ref[...]
    pT = posT_ref[...]
    sq_r = sqr_ref[...]
    sq_c = sqc_ref[...]
    aug = aug_ref[...]
    aug_hi = aug.astype(jnp.bfloat16).astype(jnp.float32)
    r1 = aug - aug_hi
    aug_mid = r1.astype(jnp.bfloat16).astype(jnp.float32)
    aug_lo = r1 - aug_mid
    mm = jnp.dot(pr, pT, preferred_element_type=jnp.float32)
    d = sq_r + sq_c - 2.0 * mm
    iota = lax.broadcasted_iota(jnp.int32, (RB, P), 1)
    INF = jnp.float32(jnp.inf)
    BIGI = jnp.int32(2**30)
    cols = []
    prels = []
    for _ in range(K):
        m = jnp.min(d, axis=1, keepdims=True)
        is_min = d <= m
        idx = jnp.min(jnp.where(is_min, iota, BIGI), axis=1, keepdims=True)
        onehot = iota == idx
        ohf = onehot.astype(jnp.float32)
        sel = (jnp.dot(ohf, aug_hi, preferred_element_type=jnp.float32)
               + jnp.dot(ohf, aug_mid, preferred_element_type=jnp.float32)
               + jnp.dot(ohf, aug_lo, preferred_element_type=jnp.float32))
        cols.append(idx)
        prels.append(sel[:, 0:3] - pr[:, 0:3])
        d = jnp.where(onehot, INF, d)
    col_ref[...] = jnp.concatenate(cols, axis=1)
    prel_ref[...] = jnp.concatenate(prels, axis=1)


def _knn(sq2, sq1, posT, aug):
    nb = P // RB
    return pl.pallas_call(
        _knn_body,
        grid=(nb,),
        in_specs=[
            pl.BlockSpec((RB, 1), lambda i: (i, 0)),
            pl.BlockSpec((1, P), lambda i: (0, 0)),
            pl.BlockSpec((RB, 8), lambda i: (i, 0)),
            pl.BlockSpec((8, P), lambda i: (0, 0)),
            pl.BlockSpec((P, 8), lambda i: (0, 0)),
        ],
        out_specs=[
            pl.BlockSpec((RB, K), lambda i: (i, 0)),
            pl.BlockSpec((RB, K * D), lambda i: (i, 0)),
        ],
        out_shape=[
            jax.ShapeDtypeStruct((P, K), jnp.int32),
            jax.ShapeDtypeStruct((P, K * D), jnp.float32),
        ],
    )(sq2, sq1, aug, posT, aug)


def _sc_gather(table, idx):
    nw = _SC_CORES * _SC_SUBCORES
    b = idx.shape[0]
    b_per_w = b // nw
    width = table.shape[1]
    mesh = plsc.VectorSubcoreMesh(core_axis_name="c", subcore_axis_name="s")

    @functools.partial(
        pl.kernel,
        mesh=mesh,
        out_type=jax.ShapeDtypeStruct((b, width), jnp.float32),
        scratch_types=[
            pltpu.VMEM((_SC_CHUNK,), jnp.int32),
            pltpu.VMEM((_SC_CHUNK, width), jnp.float32),
            pltpu.SemaphoreType.DMA,
        ],
    )
    def gk(table_hbm, idx_hbm, out_hbm, idx_v, rows_v, sem):
        wid = lax.axis_index("s") * _SC_CORES + lax.axis_index("c")
        base = wid * b_per_w
        for c in range(b_per_w // _SC_CHUNK):
            off = base + c * _SC_CHUNK
            pltpu.sync_copy(idx_hbm.at[pl.ds(off, _SC_CHUNK)], idx_v)
            pltpu.async_copy(table_hbm.at[idx_v], rows_v, sem).wait()
            pltpu.sync_copy(rows_v, out_hbm.at[pl.ds(off, _SC_CHUNK)])

    return gk(table, idx)


def _elu(x):
    return jnp.where(x > 0, x, jnp.exp(jnp.where(x > 0, 0.0, x)) - 1.0)


def _bn_cols(t, g, b):
    m = jnp.mean(t, axis=0, keepdims=True)
    dev = t - m
    v = jnp.mean(dev * dev, axis=0, keepdims=True)
    return dev / jnp.sqrt(v + EPS) * g + b


def _bn_pooled(t, g, b):
    s = t[:, 0:C_DELTA]
    for l in range(1, K):
        s = s + t[:, C_DELTA * l:C_DELTA * (l + 1)]
    m16 = jnp.sum(s, axis=0, keepdims=True) / (P * K)
    m = jnp.concatenate([m16] * K, axis=1)
    dev = t - m
    s2 = dev[:, 0:C_DELTA] * dev[:, 0:C_DELTA]
    for l in range(1, K):
        dl = dev[:, C_DELTA * l:C_DELTA * (l + 1)]
        s2 = s2 + dl * dl
    v16 = jnp.sum(s2, axis=0, keepdims=True) / (P * K)
    v = jnp.concatenate([v16] * K, axis=1)
    return dev / jnp.sqrt(v + EPS) * g + b


def _dense_body(prel_ref, xj_ref, w1b_ref, b1t_ref, g1t_ref, be1t_ref,
                w2b_ref, b2t_ref, g2t_ref, be2t_ref,
                wm_ref, bm_ref, gm_ref, bem_ref,
                wc1b_ref, bc1_ref, gc1_ref, bec1_ref,
                wc2b_ref, bc2_ref, gc2_ref, bec2_ref,
                wft_ref, bf_ref, wl_ref, bl_ref, out_ref):
    prel48 = prel_ref[...]
    h = jnp.dot(prel48, w1b_ref[...], preferred_element_type=jnp.float32) + b1t_ref[...]
    h = _bn_pooled(_elu(h), g1t_ref[...], be1t_ref[...])
    h = jnp.dot(h, w2b_ref[...], preferred_element_type=jnp.float32) + b2t_ref[...]
    h = _bn_pooled(_elu(h), g2t_ref[...], be2t_ref[...])
    t0 = jnp.dot(prel48, wm_ref[...], preferred_element_type=jnp.float32) + bm_ref[...]
    t0 = _bn_cols(_elu(t0), gm_ref[...], bem_ref[...])
    t1 = jnp.dot(t0, wc1b_ref[...], preferred_element_type=jnp.float32) + bc1_ref[...]
    t1 = _bn_cols(_elu(t1), gc1_ref[...], bec1_ref[...])
    t2 = jnp.dot(t1, wc2b_ref[...], preferred_element_type=jnp.float32) + bc2_ref[...]
    t2 = _bn_cols(t2, gc2_ref[...], bec2_ref[...])
    wft = wft_ref[...]
    acc = jnp.zeros((P, C_TOT), jnp.float32)
    for l in range(K):
        u_l = jnp.dot(t2[:, C_DELTA * l:C_DELTA * (l + 1)], wft,
                      preferred_element_type=jnp.float32)
        xs_l = jnp.concatenate(
            [h[:, C_DELTA * l:C_DELTA * (l + 1)],
             xj_ref[:, C_IN * l:C_IN * (l + 1)]], axis=1)
        acc = acc + xs_l * u_l
    out = jnp.dot(acc + bf_ref[...], wl_ref[...],
                  preferred_element_type=jnp.float32) + bl_ref[...]
    out_ref[...] = out


def _dense(prel48, xj, *weights):
    return pl.pallas_call(
        _dense_body,
        out_shape=jax.ShapeDtypeStruct((P, C_OUT), jnp.float32),
    )(prel48, xj, *weights)


def kernel(x, pos, w1, b1, g1, be1, w2, b2, g2, be2, wm, bm, gm, bem,
           wc1, bc1, gc1, bec1, wc2, bc2, gc2, bec2, wf, bf, wl, bl):
    xf = x.reshape(P, C_IN)
    pf = pos.reshape(P, D)
    sq = jnp.sum(pf * pf, axis=1)
    sq2 = sq.reshape(P, 1)
    sq1 = sq.reshape(1, P)
    posT = jnp.zeros((8, P), jnp.float32).at[0:D].set(pf.T)
    aug = jnp.concatenate(
        [pf, jnp.arange(P, dtype=jnp.float32).reshape(P, 1),
         jnp.zeros((P, 4), jnp.float32)], axis=1)
    col, prel48 = _knn(sq2, sq1, posT, aug)

    xf_pad = jnp.zeros((P, 128), jnp.float32).at[:, :C_IN].set(xf)
    xj = _sc_gather(xf_pad, col.reshape(-1))[:, :C_IN]
    xj = xj.reshape(P, K * C_IN)

    eye = jnp.eye(K, dtype=jnp.float32)
    w1b = jnp.kron(eye, w1)
    w2b = jnp.kron(eye, w2)
    wc1b = jnp.einsum('glo,gh->glho', jnp.transpose(wc1, (0, 2, 1)), eye)
    wc1b = wc1b.reshape(K * K, K * K)
    wc2b = jnp.einsum('glo,gh->glho', jnp.transpose(wc2, (0, 2, 1)), eye)
    wc2b = wc2b.reshape(K * K, K * K)
    tile = lambda a: jnp.tile(a, K).reshape(1, -1)
    weights = (
        w1b, tile(b1), tile(g1), tile(be1),
        w2b, tile(b2), tile(g2), tile(be2),
        wm, bm.reshape(1, -1), gm.reshape(1, -1), bem.reshape(1, -1),
        wc1b, bc1.reshape(1, -1), gc1.reshape(1, -1), bec1.reshape(1, -1),
        wc2b, bc2.reshape(1, -1), gc2.reshape(1, -1), bec2.reshape(1, -1),
        wf.T, bf.reshape(1, -1), wl, bl.reshape(1, -1),
    )
    out = _dense(prel48, xj, *weights)
    return out.reshape(4, 1024, C_OUT)


# speedup vs baseline: 7.4350x; 1.8145x over previous
import functools

import jax
import jax.numpy as jnp
from jax import lax
from jax.experimental import pallas as pl
from jax.experimental.pallas import tpu as pltpu
from jax.experimental.pallas import tpu_sc as plsc

P = 4096
K = 16
D = 3
C_IN = 64
C_DELTA = 16
C_TOT = 80
C_OUT = 64
EPS = 1e-5
RB = 512
_SC_CORES = 2
_SC_SUBCORES = 16
_SC_CHUNK = 256


def _knn_body(sqr_ref, sqc_ref, posr_ref, posT_ref, col_ref):
    pr = posr_ref[...]
    pT = posT_ref[...]
    mm = jnp.dot(pr, pT, preferred_element_type=jnp.float32)
    d = sqr_ref[...] + sqc_ref[...] - 2.0 * mm
    iota = lax.broadcasted_iota(jnp.int32, (RB, P), 1)
    INF = jnp.float32(jnp.inf)
    BIGI = jnp.int32(2**30)
    cols = []
    for _ in range(K):
        m = jnp.min(d, axis=1, keepdims=True)
        idx = jnp.min(jnp.where(d <= m, iota, BIGI), axis=1, keepdims=True)
        cols.append(idx)
        d = jnp.where(iota == idx, INF, d)
    col_ref[...] = jnp.concatenate(cols, axis=1)


def _knn(sq2, sq1, posr, posT):
    nb = P // RB
    return pl.pallas_call(
        _knn_body,
        grid=(nb,),
        in_specs=[
            pl.BlockSpec((RB, 1), lambda i: (i, 0)),
            pl.BlockSpec((1, P), lambda i: (0, 0)),
            pl.BlockSpec((RB, 8), lambda i: (i, 0)),
            pl.BlockSpec((8, P), lambda i: (0, 0)),
        ],
        out_specs=pl.BlockSpec((RB, K), lambda i: (i, 0)),
        out_shape=jax.ShapeDtypeStruct((P, K), jnp.int32),
    )(sq2, sq1, posr, posT)


def _sc_gather(table, idx):
    nw = _SC_CORES * _SC_SUBCORES
    b = idx.shape[0]
    b_per_w = b // nw
    width = table.shape[1]
    mesh = plsc.VectorSubcoreMesh(core_axis_name="c", subcore_axis_name="s")

    @functools.partial(
        pl.kernel,
        mesh=mesh,
        out_type=jax.ShapeDtypeStruct((b, width), jnp.float32),
        scratch_types=[
            pltpu.VMEM((_SC_CHUNK,), jnp.int32),
            pltpu.VMEM((_SC_CHUNK, width), jnp.float32),
            pltpu.SemaphoreType.DMA,
        ],
    )
    def gk(table_hbm, idx_hbm, out_hbm, idx_v, rows_v, sem):
        wid = lax.axis_index("s") * _SC_CORES + lax.axis_index("c")
        base = wid * b_per_w
        for c in range(b_per_w // _SC_CHUNK):
            off = base + c * _SC_CHUNK
            pltpu.sync_copy(idx_hbm.at[pl.ds(off, _SC_CHUNK)], idx_v)
            pltpu.async_copy(table_hbm.at[idx_v], rows_v, sem).wait()
            pltpu.sync_copy(rows_v, out_hbm.at[pl.ds(off, _SC_CHUNK)])

    return gk(table, idx)


def _elu(x):
    return jnp.where(x > 0, x, jnp.exp(jnp.where(x > 0, 0.0, x)) - 1.0)


def _bn_cols(t, g, b):
    m = jnp.mean(t, axis=0, keepdims=True)
    dev = t - m
    v = jnp.mean(dev * dev, axis=0, keepdims=True)
    return dev / jnp.sqrt(v + EPS) * g + b


def _bn_pooled(t, g, b):
    s = t[:, 0:C_DELTA]
    for l in range(1, K):
        s = s + t[:, C_DELTA * l:C_DELTA * (l + 1)]
    m16 = jnp.sum(s, axis=0, keepdims=True) / (P * K)
    m = jnp.concatenate([m16] * K, axis=1)
    dev = t - m
    s2 = dev[:, 0:C_DELTA] * dev[:, 0:C_DELTA]
    for l in range(1, K):
        dl = dev[:, C_DELTA * l:C_DELTA * (l + 1)]
        s2 = s2 + dl * dl
    v16 = jnp.sum(s2, axis=0, keepdims=True) / (P * K)
    v = jnp.concatenate([v16] * K, axis=1)
    return dev / jnp.sqrt(v + EPS) * g + b


def _dense_body(pn_ref, pt_ref, xj_ref, w1b_ref, b1t_ref, g1t_ref, be1t_ref,
                w2b_ref, b2t_ref, g2t_ref, be2t_ref,
                wm_ref, bm_ref, gm_ref, bem_ref,
                wc1b_ref, bc1_ref, gc1_ref, bec1_ref,
                wc2b_ref, bc2_ref, gc2_ref, bec2_ref,
                wft_ref, bf_ref, wl_ref, bl_ref, out_ref):
    prel48 = pn_ref[...] - pt_ref[...]
    h = jnp.dot(prel48, w1b_ref[...], preferred_element_type=jnp.float32) + b1t_ref[...]
    h = _bn_pooled(_elu(h), g1t_ref[...], be1t_ref[...])
    h = jnp.dot(h, w2b_ref[...], preferred_element_type=jnp.float32) + b2t_ref[...]
    h = _bn_pooled(_elu(h), g2t_ref[...], be2t_ref[...])
    t0 = jnp.dot(prel48, wm_ref[...], preferred_element_type=jnp.float32) + bm_ref[...]
    t0 = _bn_cols(_elu(t0), gm_ref[...], bem_ref[...])
    t1 = jnp.dot(t0, wc1b_ref[...], preferred_element_type=jnp.float32) + bc1_ref[...]
    t1 = _bn_cols(_elu(t1), gc1_ref[...], bec1_ref[...])
    t2 = jnp.dot(t1, wc2b_ref[...], preferred_element_type=jnp.float32) + bc2_ref[...]
    t2 = _bn_cols(t2, gc2_ref[...], bec2_ref[...])
    wft = wft_ref[...]
    acc = jnp.zeros((P, C_TOT), jnp.float32)
    for l in range(K):
        u_l = jnp.dot(t2[:, C_DELTA * l:C_DELTA * (l + 1)], wft,
                      preferred_element_type=jnp.float32)
        xs_l = jnp.concatenate(
            [h[:, C_DELTA * l:C_DELTA * (l + 1)],
             xj_ref[:, C_IN * l:C_IN * (l + 1)]], axis=1)
        acc = acc + xs_l * u_l
    out = jnp.dot(acc + bf_ref[...], wl_ref[...],
                  preferred_element_type=jnp.float32) + bl_ref[...]
    out_ref[...] = out


def _dense(pn48, pt48, xj, *weights):
    return pl.pallas_call(
        _dense_body,
        out_shape=jax.ShapeDtypeStruct((P, C_OUT), jnp.float32),
    )(pn48, pt48, xj, *weights)


def kernel(x, pos, w1, b1, g1, be1, w2, b2, g2, be2, wm, bm, gm, bem,
           wc1, bc1, gc1, bec1, wc2, bc2, gc2, bec2, wf, bf, wl, bl):
    xf = x.reshape(P, C_IN)
    pf = pos.reshape(P, D)
    sq = jnp.sum(pf * pf, axis=1)
    sq2 = sq.reshape(P, 1)
    sq1 = sq.reshape(1, P)
    posr = jnp.zeros((P, 8), jnp.float32).at[:, 0:D].set(pf)
    posT = jnp.zeros((8, P), jnp.float32).at[0:D].set(pf.T)
    col = _knn(sq2, sq1, posr, posT)

    table = jnp.zeros((P, 128), jnp.float32).at[:, :C_IN].set(xf)
    table = table.at[:, C_IN:C_IN + D].set(pf)
    g = _sc_gather(table, col.reshape(-1)).reshape(P, K, 128)
    xj = g[:, :, :C_IN].reshape(P, K * C_IN)
    pn48 = g[:, :, C_IN:C_IN + D].reshape(P, K * D)
    pt48 = jnp.tile(pf, (1, K))

    eye = jnp.eye(K, dtype=jnp.float32)
    w1b = jnp.kron(eye, w1)
    w2b = jnp.kron(eye, w2)
    wc1b = jnp.einsum('glo,gh->glho', jnp.transpose(wc1, (0, 2, 1)), eye)
    wc1b = wc1b.reshape(K * K, K * K)
    wc2b = jnp.einsum('glo,gh->glho', jnp.transpose(wc2, (0, 2, 1)), eye)
    wc2b = wc2b.reshape(K * K, K * K)
    tile = lambda a: jnp.tile(a, K).reshape(1, -1)
    weights = (
        w1b, tile(b1), tile(g1), tile(be1),
        w2b, tile(b2), tile(g2), tile(be2),
        wm, bm.reshape(1, -1), gm.reshape(1, -1), bem.reshape(1, -1),
        wc1b, bc1.reshape(1, -1), gc1.reshape(1, -1), bec1.reshape(1, -1),
        wc2b, bc2.reshape(1, -1), gc2.reshape(1, -1), bec2.reshape(1, -1),
        wf.T, bf.reshape(1, -1), wl, bl.reshape(1, -1),
    )
    out = _dense(pn48, pt48, xj, *weights)
    return out.reshape(4, 1024, C_OUT)


# speedup vs baseline: 7.4411x; 1.0008x over previous
import functools

import jax
import jax.numpy as jnp
from jax import lax
from jax.experimental import pallas as pl
from jax.experimental.pallas import tpu as pltpu
from jax.experimental.pallas import tpu_sc as plsc

P = 4096
K = 16
D = 3
C_IN = 64
C_DELTA = 16
C_TOT = 80
C_OUT = 64
EPS = 1e-5
RB = 512
_SC_CORES = 2
_SC_SUBCORES = 16
_SC_CHUNK = 256


def _knn_body(sqr_ref, sqc_ref, posr_ref, posT_ref, col_ref):
    pr = posr_ref[...]
    pT = posT_ref[...]
    mm = jnp.dot(pr, pT, preferred_element_type=jnp.float32)
    d = sqr_ref[...] + sqc_ref[...] - 2.0 * mm
    iota = lax.broadcasted_iota(jnp.int32, (RB, P), 1)
    iota128 = lax.broadcasted_iota(jnp.int32, (RB, 128), 1)
    INF = jnp.float32(jnp.inf)
    BIGI = jnp.int32(2**30)
    cols = []
    for _ in range(K):
        vs = [d[:, 128 * c:128 * (c + 1)] for c in range(P // 128)]
        cs = [iota128 + 128 * c for c in range(P // 128)]
        while len(vs) > 1:
            nv, nc = [], []
            for a in range(0, len(vs), 2):
                va, vb = vs[a], vs[a + 1]
                le = va <= vb
                nv.append(jnp.minimum(va, vb))
                nc.append(jnp.where(le, cs[a], cs[a + 1]))
            vs, cs = nv, nc
        v, ci = vs[0], cs[0]
        m = jnp.min(v, axis=1, keepdims=True)
        idx = jnp.min(jnp.where(v <= m, ci, BIGI), axis=1, keepdims=True)
        cols.append(idx)
        d = jnp.where(iota == idx, INF, d)
    col_ref[...] = jnp.concatenate(cols, axis=1)


def _knn(sq2, sq1, posr, posT):
    nb = P // RB
    return pl.pallas_call(
        _knn_body,
        grid=(nb,),
        in_specs=[
            pl.BlockSpec((RB, 1), lambda i: (i, 0)),
            pl.BlockSpec((1, P), lambda i: (0, 0)),
            pl.BlockSpec((RB, 8), lambda i: (i, 0)),
            pl.BlockSpec((8, P), lambda i: (0, 0)),
        ],
        out_specs=pl.BlockSpec((RB, K), lambda i: (i, 0)),
        out_shape=jax.ShapeDtypeStruct((P, K), jnp.int32),
    )(sq2, sq1, posr, posT)


def _sc_gather(table, idx):
    nw = _SC_CORES * _SC_SUBCORES
    b = idx.shape[0]
    b_per_w = b // nw
    width = table.shape[1]
    mesh = plsc.VectorSubcoreMesh(core_axis_name="c", subcore_axis_name="s")

    @functools.partial(
        pl.kernel,
        mesh=mesh,
        out_type=jax.ShapeDtypeStruct((b, width), jnp.float32),
        scratch_types=[
            pltpu.VMEM((_SC_CHUNK,), jnp.int32),
            pltpu.VMEM((_SC_CHUNK, width), jnp.float32),
            pltpu.SemaphoreType.DMA,
        ],
    )
    def gk(table_hbm, idx_hbm, out_hbm, idx_v, rows_v, sem):
        wid = lax.axis_index("s") * _SC_CORES + lax.axis_index("c")
        base = wid * b_per_w
        for c in range(b_per_w // _SC_CHUNK):
            off = base + c * _SC_CHUNK
            pltpu.sync_copy(idx_hbm.at[pl.ds(off, _SC_CHUNK)], idx_v)
            pltpu.async_copy(table_hbm.at[idx_v], rows_v, sem).wait()
            pltpu.sync_copy(rows_v, out_hbm.at[pl.ds(off, _SC_CHUNK)])

    return gk(table, idx)


def _elu(x):
    return jnp.where(x > 0, x, jnp.exp(jnp.where(x > 0, 0.0, x)) - 1.0)


def _bn_cols(t, g, b):
    m = jnp.mean(t, axis=0, keepdims=True)
    dev = t - m
    v = jnp.mean(dev * dev, axis=0, keepdims=True)
    return dev / jnp.sqrt(v + EPS) * g + b


def _bn_pooled(t, g, b):
    s = t[:, 0:C_DELTA]
    for l in range(1, K):
        s = s + t[:, C_DELTA * l:C_DELTA * (l + 1)]
    m16 = jnp.sum(s, axis=0, keepdims=True) / (P * K)
    m = jnp.concatenate([m16] * K, axis=1)
    dev = t - m
    s2 = dev[:, 0:C_DELTA] * dev[:, 0:C_DELTA]
    for l in range(1, K):
        dl = dev[:, C_DELTA * l:C_DELTA * (l + 1)]
        s2 = s2 + dl * dl
    v16 = jnp.sum(s2, axis=0, keepdims=True) / (P * K)
    v = jnp.concatenate([v16] * K, axis=1)
    return dev / jnp.sqrt(v + EPS) * g + b


def _dense1_body(pn_ref, pt_ref, xj_ref, w1b_ref, b1t_ref, g1t_ref, be1t_ref,
                 w2b_ref, b2t_ref, g2t_ref, be2t_ref,
                 wm_ref, bm_ref, gm_ref, bem_ref,
                 wc1b_ref, bc1_ref, gc1_ref, bec1_ref,
                 wc2b_ref, bc2_ref, gc2_ref, bec2_ref,
                 wft_ref, bf_ref, wl_ref, bl_ref, out_ref):
    prel48 = pn_ref[...] - pt_ref[...]
    h = jnp.dot(prel48, w1b_ref[...], preferred_element_type=jnp.float32) + b1t_ref[...]
    h = _bn_pooled(_elu(h), g1t_ref[...], be1t_ref[...])
    h = jnp.dot(h, w2b_ref[...], preferred_element_type=jnp.float32) + b2t_ref[...]
    h = _bn_pooled(_elu(h), g2t_ref[...], be2t_ref[...])
    t0 = jnp.dot(prel48, wm_ref[...], preferred_element_type=jnp.float32) + bm_ref[...]
    t0 = _bn_cols(_elu(t0), gm_ref[...], bem_ref[...])
    t1 = jnp.dot(t0, wc1b_ref[...], preferred_element_type=jnp.float32) + bc1_ref[...]
    t1 = _bn_cols(_elu(t1), gc1_ref[...], bec1_ref[...])
    t2 = jnp.dot(t1, wc2b_ref[...], preferred_element_type=jnp.float32) + bc2_ref[...]
    t2 = _bn_cols(t2, gc2_ref[...], bec2_ref[...])
    wft = wft_ref[...]
    acc = jnp.zeros((P, C_TOT), jnp.float32)
    for l in range(K):
        u_l = jnp.dot(t2[:, C_DELTA * l:C_DELTA * (l + 1)], wft,
                      preferred_element_type=jnp.float32)
        xs_l = jnp.concatenate(
            [h[:, C_DELTA * l:C_DELTA * (l + 1)],
             xj_ref[:, C_IN * l:C_IN * (l + 1)]], axis=1)
        acc = acc + xs_l * u_l
    out = jnp.dot(acc + bf_ref[...], wl_ref[...],
                  preferred_element_type=jnp.float32) + bl_ref[...]
    out_ref[...] = out


def _dense1(pn48, pt48, xj, *weights):
    return pl.pallas_call(
        _dense1_body,
        out_shape=jax.ShapeDtypeStruct((P, C_OUT), jnp.float32),
    )(pn48, pt48, xj, *weights)


def kernel(x, pos, w1, b1, g1, be1, w2, b2, g2, be2, wm, bm, gm, bem,
           wc1, bc1, gc1, bec1, wc2, bc2, gc2, bec2, wf, bf, wl, bl):
    xf = x.reshape(P, C_IN)
    pf = pos.reshape(P, D)
    sq = jnp.sum(pf * pf, axis=1)
    sq2 = sq.reshape(P, 1)
    sq1 = sq.reshape(1, P)
    posr = jnp.zeros((P, 8), jnp.float32).at[:, 0:D].set(pf)
    posT = jnp.zeros((8, P), jnp.float32).at[0:D].set(pf.T)
    col = _knn(sq2, sq1, posr, posT)

    table = jnp.zeros((P, 128), jnp.float32).at[:, :C_IN].set(xf)
    table = table.at[:, C_IN:C_IN + D].set(pf)
    g3 = _sc_gather(table, col.reshape(-1)).reshape(P, K, 128)
    xj = g3[:, :, :C_IN].reshape(P, K * C_IN)
    pn48 = g3[:, :, C_IN:C_IN + D].reshape(P, K * D)
    pt48 = jnp.tile(pf, (1, K))

    eye = jnp.eye(K, dtype=jnp.float32)
    w1b = jnp.kron(eye, w1)
    w2b = jnp.kron(eye, w2)
    wc1b = jnp.einsum('glo,gh->glho', jnp.transpose(wc1, (0, 2, 1)), eye)
    wc1b = wc1b.reshape(K * K, K * K)
    wc2b = jnp.einsum('glo,gh->glho', jnp.transpose(wc2, (0, 2, 1)), eye)
    wc2b = wc2b.reshape(K * K, K * K)
    tile = lambda a: jnp.tile(a, K).reshape(1, -1)
    weights = (
        w1b, tile(b1), tile(g1), tile(be1),
        w2b, tile(b2), tile(g2), tile(be2),
        wm, bm.reshape(1, -1), gm.reshape(1, -1), bem.reshape(1, -1),
        wc1b, bc1.reshape(1, -1), gc1.reshape(1, -1), bec1.reshape(1, -1),
        wc2b, bc2.reshape(1, -1), gc2.reshape(1, -1), bec2.reshape(1, -1),
        wf.T, bf.reshape(1, -1), wl, bl.reshape(1, -1),
    )
    out = _dense1(pn48, pt48, xj, *weights)
    return out.reshape(4, 1024, C_OUT)


# speedup vs baseline: 7.5138x; 1.0098x over previous
import functools

import jax
import jax.numpy as jnp
from jax import lax
from jax.experimental import pallas as pl
from jax.experimental.pallas import tpu as pltpu
from jax.experimental.pallas import tpu_sc as plsc

P = 4096
K = 16
D = 3
C_IN = 64
C_DELTA = 16
C_TOT = 80
C_OUT = 64
EPS = 1e-5
RB = 512
_SC_CORES = 2
_SC_SUBCORES = 16
_SC_CHUNK = 512


def _knn_body(sqr_ref, sqc_ref, posr_ref, posT_ref, col_ref):
    pr = posr_ref[...]
    pT = posT_ref[...]
    mm = jnp.dot(pr, pT, preferred_element_type=jnp.float32)
    d = sqr_ref[...] + sqc_ref[...] - 2.0 * mm
    iota = lax.broadcasted_iota(jnp.int32, (RB, P), 1)
    iota128 = lax.broadcasted_iota(jnp.int32, (RB, 128), 1)
    INF = jnp.float32(jnp.inf)
    BIGI = jnp.int32(2**30)
    cols = []
    for _ in range(K):
        vs = [d[:, 128 * c:128 * (c + 1)] for c in range(P // 128)]
        cs = [iota128 + 128 * c for c in range(P // 128)]
        while len(vs) > 1:
            nv, nc = [], []
            for a in range(0, len(vs), 2):
                va, vb = vs[a], vs[a + 1]
                le = va <= vb
                nv.append(jnp.minimum(va, vb))
                nc.append(jnp.where(le, cs[a], cs[a + 1]))
            vs, cs = nv, nc
        v, ci = vs[0], cs[0]
        m = jnp.min(v, axis=1, keepdims=True)
        idx = jnp.min(jnp.where(v <= m, ci, BIGI), axis=1, keepdims=True)
        cols.append(idx)
        d = jnp.where(iota == idx, INF, d)
    col_ref[...] = jnp.concatenate(cols, axis=1)


def _knn(sq2, sq1, posr, posT):
    nb = P // RB
    return pl.pallas_call(
        _knn_body,
        grid=(nb,),
        in_specs=[
            pl.BlockSpec((RB, 1), lambda i: (i, 0)),
            pl.BlockSpec((1, P), lambda i: (0, 0)),
            pl.BlockSpec((RB, 8), lambda i: (i, 0)),
            pl.BlockSpec((8, P), lambda i: (0, 0)),
        ],
        out_specs=pl.BlockSpec((RB, K), lambda i: (i, 0)),
        out_shape=jax.ShapeDtypeStruct((P, K), jnp.int32),
    )(sq2, sq1, posr, posT)


def _sc_gather(table, idx):
    nw = _SC_CORES * _SC_SUBCORES
    b = idx.shape[0]
    b_per_w = b // nw
    width = table.shape[1]
    mesh = plsc.VectorSubcoreMesh(core_axis_name="c", subcore_axis_name="s")

    @functools.partial(
        pl.kernel,
        mesh=mesh,
        out_type=jax.ShapeDtypeStruct((b, width), jnp.float32),
        scratch_types=[
            pltpu.VMEM((_SC_CHUNK,), jnp.int32),
            pltpu.VMEM((_SC_CHUNK, width), jnp.float32),
            pltpu.SemaphoreType.DMA,
        ],
    )
    def gk(table_hbm, idx_hbm, out_hbm, idx_v, rows_v, sem):
        wid = lax.axis_index("s") * _SC_CORES + lax.axis_index("c")
        base = wid * b_per_w
        for c in range(b_per_w // _SC_CHUNK):
            off = base + c * _SC_CHUNK
            pltpu.sync_copy(idx_hbm.at[pl.ds(off, _SC_CHUNK)], idx_v)
            pltpu.async_copy(table_hbm.at[idx_v], rows_v, sem).wait()
            pltpu.sync_copy(rows_v, out_hbm.at[pl.ds(off, _SC_CHUNK)])

    return gk(table, idx)


def _elu(x):
    return jnp.where(x > 0, x, jnp.exp(jnp.where(x > 0, 0.0, x)) - 1.0)


def _bn_cols(t, g, b):
    m = jnp.mean(t, axis=0, keepdims=True)
    dev = t - m
    v = jnp.mean(dev * dev, axis=0, keepdims=True)
    return dev / jnp.sqrt(v + EPS) * g + b


def _bn_pooled(t, g, b):
    s = t[:, 0:C_DELTA]
    for l in range(1, K):
        s = s + t[:, C_DELTA * l:C_DELTA * (l + 1)]
    m16 = jnp.sum(s, axis=0, keepdims=True) / (P * K)
    m = jnp.concatenate([m16] * K, axis=1)
    dev = t - m
    s2 = dev[:, 0:C_DELTA] * dev[:, 0:C_DELTA]
    for l in range(1, K):
        dl = dev[:, C_DELTA * l:C_DELTA * (l + 1)]
        s2 = s2 + dl * dl
    v16 = jnp.sum(s2, axis=0, keepdims=True) / (P * K)
    v = jnp.concatenate([v16] * K, axis=1)
    return dev / jnp.sqrt(v + EPS) * g + b


def _dense1_body(pn_ref, pt_ref, xj_ref, w1b_ref, b1t_ref, g1t_ref, be1t_ref,
                 w2b_ref, b2t_ref, g2t_ref, be2t_ref,
                 wm_ref, bm_ref, gm_ref, bem_ref,
                 wc1b_ref, bc1_ref, gc1_ref, bec1_ref,
                 wc2b_ref, bc2_ref, gc2_ref, bec2_ref,
                 wft_ref, bf_ref, wl_ref, bl_ref, out_ref):
    prel48 = pn_ref[...] - pt_ref[...]
    h = jnp.dot(prel48, w1b_ref[...], preferred_element_type=jnp.float32) + b1t_ref[...]
    h = _bn_pooled(_elu(h), g1t_ref[...], be1t_ref[...])
    h = jnp.dot(h, w2b_ref[...], preferred_element_type=jnp.float32) + b2t_ref[...]
    h = _bn_pooled(_elu(h), g2t_ref[...], be2t_ref[...])
    t0 = jnp.dot(prel48, wm_ref[...], preferred_element_type=jnp.float32) + bm_ref[...]
    t0 = _bn_cols(_elu(t0), gm_ref[...], bem_ref[...])
    t1 = jnp.dot(t0, wc1b_ref[...], preferred_element_type=jnp.float32) + bc1_ref[...]
    t1 = _bn_cols(_elu(t1), gc1_ref[...], bec1_ref[...])
    t2 = jnp.dot(t1, wc2b_ref[...], preferred_element_type=jnp.float32) + bc2_ref[...]
    t2 = _bn_cols(t2, gc2_ref[...], bec2_ref[...])
    wft = wft_ref[...]
    acc = jnp.zeros((P, C_TOT), jnp.float32)
    for l in range(K):
        u_l = jnp.dot(t2[:, C_DELTA * l:C_DELTA * (l + 1)], wft,
                      preferred_element_type=jnp.float32)
        xs_l = jnp.concatenate(
            [h[:, C_DELTA * l:C_DELTA * (l + 1)],
             xj_ref[:, C_IN * l:C_IN * (l + 1)]], axis=1)
        acc = acc + xs_l * u_l
    out = jnp.dot(acc + bf_ref[...], wl_ref[...],
                  preferred_element_type=jnp.float32) + bl_ref[...]
    out_ref[...] = out


def _dense1(pn48, pt48, xj, *weights):
    return pl.pallas_call(
        _dense1_body,
        out_shape=jax.ShapeDtypeStruct((P, C_OUT), jnp.float32),
    )(pn48, pt48, xj, *weights)


def kernel(x, pos, w1, b1, g1, be1, w2, b2, g2, be2, wm, bm, gm, bem,
           wc1, bc1, gc1, bec1, wc2, bc2, gc2, bec2, wf, bf, wl, bl):
    xf = x.reshape(P, C_IN)
    pf = pos.reshape(P, D)
    sq = jnp.sum(pf * pf, axis=1)
    sq2 = sq.reshape(P, 1)
    sq1 = sq.reshape(1, P)
    posr = jnp.zeros((P, 8), jnp.float32).at[:, 0:D].set(pf)
    posT = jnp.zeros((8, P), jnp.float32).at[0:D].set(pf.T)
    col = _knn(sq2, sq1, posr, posT)

    table = jnp.zeros((P, 128), jnp.float32).at[:, :C_IN].set(xf)
    table = table.at[:, C_IN:C_IN + D].set(pf)
    g3 = _sc_gather(table, col.reshape(-1)).reshape(P, K, 128)
    xj = g3[:, :, :C_IN].reshape(P, K * C_IN)
    pn48 = g3[:, :, C_IN:C_IN + D].reshape(P, K * D)
    pt48 = jnp.tile(pf, (1, K))

    eye = jnp.eye(K, dtype=jnp.float32)
    w1b = jnp.kron(eye, w1)
    w2b = jnp.kron(eye, w2)
    wc1b = jnp.einsum('glo,gh->glho', jnp.transpose(wc1, (0, 2, 1)), eye)
    wc1b = wc1b.reshape(K * K, K * K)
    wc2b = jnp.einsum('glo,gh->glho', jnp.transpose(wc2, (0, 2, 1)), eye)
    wc2b = wc2b.reshape(K * K, K * K)
    tile = lambda a: jnp.tile(a, K).reshape(1, -1)
    weights = (
        w1b, tile(b1), tile(g1), tile(be1),
        w2b, tile(b2), tile(g2), tile(be2),
        wm, bm.reshape(1, -1), gm.reshape(1, -1), bem.reshape(1, -1),
        wc1b, bc1.reshape(1, -1), gc1.reshape(1, -1), bec1.reshape(1, -1),
        wc2b, bc2.reshape(1, -1), gc2.reshape(1, -1), bec2.reshape(1, -1),
        wf.T, bf.reshape(1, -1), wl, bl.reshape(1, -1),
    )
    out = _dense1(pn48, pt48, xj, *weights)
    return out.reshape(4, 1024, C_OUT)


# speedup vs baseline: 7.5164x; 1.0004x over previous
import functools

import jax
import jax.numpy as jnp
from jax import lax
from jax.experimental import pallas as pl
from jax.experimental.pallas import tpu as pltpu
from jax.experimental.pallas import tpu_sc as plsc

P = 4096
K = 16
D = 3
C_IN = 64
C_DELTA = 16
C_TOT = 80
C_OUT = 64
EPS = 1e-5
RB = 256
_SC_CORES = 2
_SC_SUBCORES = 16
_SC_CHUNK = 512


def _knn_body(sqr_ref, sqc_ref, posr_ref, posT_ref, col_ref):
    pr = posr_ref[...]
    pT = posT_ref[...]
    mm = jnp.dot(pr, pT, preferred_element_type=jnp.float32)
    d = sqr_ref[...] + sqc_ref[...] - 2.0 * mm
    iota = lax.broadcasted_iota(jnp.int32, (RB, P), 1)
    iota128 = lax.broadcasted_iota(jnp.int32, (RB, 128), 1)
    INF = jnp.float32(jnp.inf)
    BIGI = jnp.int32(2**30)
    cols = []
    for _ in range(K):
        vs = [d[:, 128 * c:128 * (c + 1)] for c in range(P // 128)]
        cs = [iota128 + 128 * c for c in range(P // 128)]
        while len(vs) > 1:
            nv, nc = [], []
            for a in range(0, len(vs), 2):
                va, vb = vs[a], vs[a + 1]
                le = va <= vb
                nv.append(jnp.minimum(va, vb))
                nc.append(jnp.where(le, cs[a], cs[a + 1]))
            vs, cs = nv, nc
        v, ci = vs[0], cs[0]
        m = jnp.min(v, axis=1, keepdims=True)
        idx = jnp.min(jnp.where(v <= m, ci, BIGI), axis=1, keepdims=True)
        cols.append(idx)
        d = jnp.where(iota == idx, INF, d)
    col_ref[...] = jnp.concatenate(cols, axis=1)


def _knn(sq2, sq1, posr, posT):
    nb = P // RB
    return pl.pallas_call(
        _knn_body,
        grid=(nb,),
        in_specs=[
            pl.BlockSpec((RB, 1), lambda i: (i, 0)),
            pl.BlockSpec((1, P), lambda i: (0, 0)),
            pl.BlockSpec((RB, 8), lambda i: (i, 0)),
            pl.BlockSpec((8, P), lambda i: (0, 0)),
        ],
        out_specs=pl.BlockSpec((RB, K), lambda i: (i, 0)),
        out_shape=jax.ShapeDtypeStruct((P, K), jnp.int32),
    )(sq2, sq1, posr, posT)


def _sc_gather(table, idx):
    nw = _SC_CORES * _SC_SUBCORES
    b = idx.shape[0]
    b_per_w = b // nw
    width = table.shape[1]
    mesh = plsc.VectorSubcoreMesh(core_axis_name="c", subcore_axis_name="s")

    @functools.partial(
        pl.kernel,
        mesh=mesh,
        out_type=jax.ShapeDtypeStruct((b, width), jnp.float32),
        scratch_types=[
            pltpu.VMEM((_SC_CHUNK,), jnp.int32),
            pltpu.VMEM((_SC_CHUNK, width), jnp.float32),
            pltpu.SemaphoreType.DMA,
        ],
    )
    def gk(table_hbm, idx_hbm, out_hbm, idx_v, rows_v, sem):
        wid = lax.axis_index("s") * _SC_CORES + lax.axis_index("c")
        base = wid * b_per_w
        for c in range(b_per_w // _SC_CHUNK):
            off = base + c * _SC_CHUNK
            pltpu.sync_copy(idx_hbm.at[pl.ds(off, _SC_CHUNK)], idx_v)
            pltpu.async_copy(table_hbm.at[idx_v], rows_v, sem).wait()
            pltpu.sync_copy(rows_v, out_hbm.at[pl.ds(off, _SC_CHUNK)])

    return gk(table, idx)


def _elu(x):
    return jnp.where(x > 0, x, jnp.exp(jnp.where(x > 0, 0.0, x)) - 1.0)


def _bn_cols(t, g, b):
    m = jnp.mean(t, axis=0, keepdims=True)
    dev = t - m
    v = jnp.mean(dev * dev, axis=0, keepdims=True)
    return dev / jnp.sqrt(v + EPS) * g + b


def _bn_pooled(t, g, b):
    s = t[:, 0:C_DELTA]
    for l in range(1, K):
        s = s + t[:, C_DELTA * l:C_DELTA * (l + 1)]
    m16 = jnp.sum(s, axis=0, keepdims=True) / (P * K)
    m = jnp.concatenate([m16] * K, axis=1)
    dev = t - m
    s2 = dev[:, 0:C_DELTA] * dev[:, 0:C_DELTA]
    for l in range(1, K):
        dl = dev[:, C_DELTA * l:C_DELTA * (l + 1)]
        s2 = s2 + dl * dl
    v16 = jnp.sum(s2, axis=0, keepdims=True) / (P * K)
    v = jnp.concatenate([v16] * K, axis=1)
    return dev / jnp.sqrt(v + EPS) * g + b


def _dense1_body(pn_ref, pt_ref, xj_ref, w1b_ref, b1t_ref, g1t_ref, be1t_ref,
                 w2b_ref, b2t_ref, g2t_ref, be2t_ref,
                 wm_ref, bm_ref, gm_ref, bem_ref,
                 wc1b_ref, bc1_ref, gc1_ref, bec1_ref,
                 wc2b_ref, bc2_ref, gc2_ref, bec2_ref,
                 wft_ref, bf_ref, wl_ref, bl_ref, out_ref):
    prel48 = pn_ref[...] - pt_ref[...]
    h = jnp.dot(prel48, w1b_ref[...], preferred_element_type=jnp.float32) + b1t_ref[...]
    h = _bn_pooled(_elu(h), g1t_ref[...], be1t_ref[...])
    h = jnp.dot(h, w2b_ref[...], preferred_element_type=jnp.float32) + b2t_ref[...]
    h = _bn_pooled(_elu(h), g2t_ref[...], be2t_ref[...])
    t0 = jnp.dot(prel48, wm_ref[...], preferred_element_type=jnp.float32) + bm_ref[...]
    t0 = _bn_cols(_elu(t0), gm_ref[...], bem_ref[...])
    t1 = jnp.dot(t0, wc1b_ref[...], preferred_element_type=jnp.float32) + bc1_ref[...]
    t1 = _bn_cols(_elu(t1), gc1_ref[...], bec1_ref[...])
    t2 = jnp.dot(t1, wc2b_ref[...], preferred_element_type=jnp.float32) + bc2_ref[...]
    t2 = _bn_cols(t2, gc2_ref[...], bec2_ref[...])
    wft = wft_ref[...]
    acc = jnp.zeros((P, C_TOT), jnp.float32)
    for l in range(K):
        u_l = jnp.dot(t2[:, C_DELTA * l:C_DELTA * (l + 1)], wft,
                      preferred_element_type=jnp.float32)
        xs_l = jnp.concatenate(
            [h[:, C_DELTA * l:C_DELTA * (l + 1)],
             xj_ref[:, C_IN * l:C_IN * (l + 1)]], axis=1)
        acc = acc + xs_l * u_l
    out = jnp.dot(acc + bf_ref[...], wl_ref[...],
                  preferred_element_type=jnp.float32) + bl_ref[...]
    out_ref[...] = out


def _dense1(pn48, pt48, xj, *weights):
    return pl.pallas_call(
        _dense1_body,
        out_shape=jax.ShapeDtypeStruct((P, C_OUT), jnp.float32),
    )(pn48, pt48, xj, *weights)


def kernel(x, pos, w1, b1, g1, be1, w2, b2, g2, be2, wm, bm, gm, bem,
           wc1, bc1, gc1, bec1, wc2, bc2, gc2, bec2, wf, bf, wl, bl):
    xf = x.reshape(P, C_IN)
    pf = pos.reshape(P, D)
    sq = jnp.sum(pf * pf, axis=1)
    sq2 = sq.reshape(P, 1)
    sq1 = sq.reshape(1, P)
    posr = jnp.zeros((P, 8), jnp.float32).at[:, 0:D].set(pf)
    posT = jnp.zeros((8, P), jnp.float32).at[0:D].set(pf.T)
    col = _knn(sq2, sq1, posr, posT)

    table = jnp.zeros((P, 128), jnp.float32).at[:, :C_IN].set(xf)
    table = table.at[:, C_IN:C_IN + D].set(pf)
    g3 = _sc_gather(table, col.reshape(-1)).reshape(P, K, 128)
    xj = g3[:, :, :C_IN].reshape(P, K * C_IN)
    pn48 = g3[:, :, C_IN:C_IN + D].reshape(P, K * D)
    pt48 = jnp.tile(pf, (1, K))

    eye = jnp.eye(K, dtype=jnp.float32)
    w1b = jnp.kron(eye, w1)
    w2b = jnp.kron(eye, w2)
    wc1b = jnp.einsum('glo,gh->glho', jnp.transpose(wc1, (0, 2, 1)), eye)
    wc1b = wc1b.reshape(K * K, K * K)
    wc2b = jnp.einsum('glo,gh->glho', jnp.transpose(wc2, (0, 2, 1)), eye)
    wc2b = wc2b.reshape(K * K, K * K)
    tile = lambda a: jnp.tile(a, K).reshape(1, -1)
    weights = (
        w1b, tile(b1), tile(g1), tile(be1),
        w2b, tile(b2), tile(g2), tile(be2),
        wm, bm.reshape(1, -1), gm.reshape(1, -1), bem.reshape(1, -1),
        wc1b, bc1.reshape(1, -1), gc1.reshape(1, -1), bec1.reshape(1, -1),
        wc2b, bc2.reshape(1, -1), gc2.reshape(1, -1), bec2.reshape(1, -1),
        wf.T, bf.reshape(1, -1), wl, bl.reshape(1, -1),
    )
    out = _dense1(pn48, pt48, xj, *weights)
    return out.reshape(4, 1024, C_OUT)
